# baseline (speedup 1.0000x reference)
import functools
import jax
import jax.numpy as jnp
from jax import lax
from jax.experimental import pallas as pl
from jax.experimental.pallas import tpu as pltpu
from jax.experimental.pallas import tpu_sc as plsc

F32 = jnp.float32
LN_EPS = 1e-12
B = 2048
EMB = 256
HID = 768
NPOS = 512
NSPAN = 30
SPAN_PAD = 32
NW = 32
BPW = B // NW
BLK = 1024

@functools.cache
def _sc_gather_counts_fn():
    mesh = plsc.VectorSubcoreMesh(core_axis_name="c", subcore_axis_name="s")
    return functools.partial(
        pl.kernel,
        out_type=[
            jax.ShapeDtypeStruct((B, EMB), F32),
            jax.ShapeDtypeStruct((B, NPOS), F32),
        ],
        mesh=mesh,
        compiler_params=pltpu.CompilerParams(needs_layout_passes=False,
                                             use_tc_tiling_on_sc=True),
        scratch_types=[
            pltpu.VMEM((BPW,), jnp.int32),
            pltpu.VMEM((BPW, EMB), F32),
            pltpu.VMEM((BPW * NSPAN + 16,), jnp.int32),
            pltpu.VMEM((BPW, NPOS), F32),
            pltpu.SemaphoreType.DMA,
        ],
    )(_sc_gather_counts)


def _sc_gather_counts(table_hbm, eid_hbm, pid_hbm, ent_out, cnt_out,
                      idx_v, rows_v, pid_v, cnt_v, sem):
    wid = lax.axis_index("s") * 2 + lax.axis_index("c")
    base = wid * BPW
    pltpu.sync_copy(eid_hbm.at[pl.ds(base, BPW)], idx_v)
    gat = pltpu.async_copy(table_hbm.at[idx_v], rows_v, sem)
    pltpu.sync_copy(pid_hbm.at[pl.ds(base * NSPAN, BPW * NSPAN)],
                    pid_v.at[pl.ds(0, BPW * NSPAN)])

    zeros16 = jnp.zeros((16,), F32)
    ones16 = jnp.ones((16,), F32)
    lane = lax.iota(jnp.int32, 16)

    def row_body(r, carry):
        def zero_body(c, carry2):
            for u in range(4):
                cnt_v[r, pl.ds(c * 64 + u * 16, 16)] = zeros16
            return carry2
        lax.fori_loop(0, NPOS // 64, zero_body, 0)
        v0 = pid_v[pl.ds(r * NSPAN, 16)]
        v1 = pid_v[pl.ds(r * NSPAN + 16, 16)]
        m0 = v0 >= 0
        m1 = (v1 >= 0) & (lane < NSPAN - 16)
        rvec = jnp.full((16,), 0, jnp.int32) + r
        plsc.addupdate_scatter(cnt_v, [rvec, v0], ones16, mask=m0)
        plsc.addupdate_scatter(cnt_v, [rvec, v1], ones16, mask=m1)
        return carry

    lax.fori_loop(0, BPW, row_body, 0)
    gat.wait()
    pltpu.sync_copy(rows_v, ent_out.at[pl.ds(base, BPW)])
    pltpu.sync_copy(cnt_v, cnt_out.at[pl.ds(base, BPW)])


def _tc_body(ent_ref, cnt_ref, tt_ref, wt_ref, pos_ref, type_ref,
             gam_ref, bet_ref, out_ref):
    cnt = cnt_ref[...]
    proj = lax.dot_general(ent_ref[...], wt_ref[...],
                           (((1,), (1,)), ((), ())),
                           preferred_element_type=F32)
    possum = jnp.dot(cnt, pos_ref[...], preferred_element_type=F32)
    n = jnp.sum(cnt, axis=-1, keepdims=True)
    pos = possum / jnp.maximum(n, 1e-7)
    t0 = type_ref[0:1, :]
    t1 = type_ref[1:2, :]
    tok = t0 + tt_ref[...].astype(F32) * (t1 - t0)
    emb = proj + pos + tok
    mu = jnp.mean(emb, axis=-1, keepdims=True)
    d = emb - mu
    var = jnp.mean(d * d, axis=-1, keepdims=True)
    out_ref[...] = d * lax.rsqrt(var + LN_EPS) * gam_ref[...] + bet_ref[...]


def _tc_call(ent, cnt, ttf, wt, pos_table, type_table, gamma, beta):
    return pl.pallas_call(
        _tc_body,
        grid=(B // BLK,),
        in_specs=[
            pl.BlockSpec((BLK, EMB), lambda i: (i, 0)),
            pl.BlockSpec((BLK, NPOS), lambda i: (i, 0)),
            pl.BlockSpec((BLK, 1), lambda i: (i, 0)),
            pl.BlockSpec((HID, EMB), lambda i: (0, 0)),
            pl.BlockSpec((NPOS, HID), lambda i: (0, 0)),
            pl.BlockSpec((2, HID), lambda i: (0, 0)),
            pl.BlockSpec((1, HID), lambda i: (0, 0)),
            pl.BlockSpec((1, HID), lambda i: (0, 0)),
        ],
        out_specs=pl.BlockSpec((BLK, HID), lambda i: (i, 0)),
        out_shape=jax.ShapeDtypeStruct((B, HID), F32),
    )(ent, cnt, ttf, wt, pos_table, type_table, gamma, beta)


def kernel(entity_ids, position_ids, token_type_ids, entity_table, dense_W,
           pos_table, type_table, gamma, beta):
    eid = entity_ids.reshape(B)
    pid_flat = position_ids.reshape(B * NSPAN)
    tti = token_type_ids.reshape(B, 1)
    ent_rows, cnt = _sc_gather_counts_fn()(entity_table, eid, pid_flat)
    out = _tc_call(ent_rows, cnt, tti, dense_W,
                   pos_table, type_table,
                   gamma.reshape(1, HID), beta.reshape(1, HID))
    return out.reshape(16, 128, HID)


# speedup vs baseline: 1.0226x; 1.0226x over previous
import functools
import jax
import jax.numpy as jnp
from jax import lax
from jax.experimental import pallas as pl
from jax.experimental.pallas import tpu as pltpu
from jax.experimental.pallas import tpu_sc as plsc

F32 = jnp.float32
LN_EPS = 1e-12
B = 2048
EMB = 256
HID = 768
NPOS = 512
NSPAN = 30
NW = 32
BPW = B // NW
BLK = 512

@functools.cache
def _sc_gather_counts_fn():
    mesh = plsc.VectorSubcoreMesh(core_axis_name="c", subcore_axis_name="s")
    return functools.partial(
        pl.kernel,
        out_type=[
            jax.ShapeDtypeStruct((B, EMB), F32),
            jax.ShapeDtypeStruct((B, NPOS), F32),
        ],
        mesh=mesh,
        compiler_params=pltpu.CompilerParams(needs_layout_passes=False,
                                             use_tc_tiling_on_sc=True),
        scratch_types=[
            pltpu.VMEM((BPW,), jnp.int32),
            pltpu.VMEM((BPW, EMB), F32),
            pltpu.VMEM((BPW, NSPAN), jnp.int32),
            pltpu.VMEM((BPW, NPOS), F32),
            pltpu.SemaphoreType.DMA,
            pltpu.SemaphoreType.DMA,
        ],
    )(_sc_gather_counts)


def _sc_gather_counts(table_hbm, eid_hbm, pid_hbm, ent_out, cnt_out,
                      idx_v, rows_v, pid_v, cnt_v, sem, sem2):
    wid = lax.axis_index("s") * 2 + lax.axis_index("c")
    base = wid * BPW
    pltpu.sync_copy(eid_hbm.at[wid // 2, pl.ds((wid % 2) * BPW, BPW)], idx_v)
    gat = pltpu.async_copy(table_hbm.at[idx_v], rows_v, sem)
    pidc = pltpu.async_copy(
        pid_hbm.at[wid // 2, pl.ds((wid % 2) * BPW, BPW), :], pid_v, sem2)

    zeros16 = jnp.zeros((16,), F32)
    ones16 = jnp.ones((16,), F32)
    lane = lax.iota(jnp.int32, 16)

    def zero_body(c, carry2):
        r = c // (NPOS // 64)
        c4 = c % (NPOS // 64)
        for u in range(4):
            cnt_v[r, pl.ds(c4 * 64 + u * 16, 16)] = zeros16
        return carry2

    lax.fori_loop(0, BPW * (NPOS // 64), zero_body, 0)
    pidc.wait()
    gat.wait()
    ent_h = pltpu.async_copy(rows_v, ent_out.at[pl.ds(base, BPW)], sem)

    def row_body(r, carry):
        v0 = pid_v[r, pl.ds(0, 16)]
        v1 = pid_v[r, pl.ds(NSPAN - 16, 16)]
        m0 = v0 >= 0
        m1 = (v1 >= 0) & (lane >= 32 - NSPAN)
        rvec = jnp.full((16,), 0, jnp.int32) + r
        plsc.addupdate_scatter(cnt_v, [rvec, v0], ones16, mask=m0)
        plsc.addupdate_scatter(cnt_v, [rvec, v1], ones16, mask=m1)
        return carry

    CH = 16
    cnt_hs = []
    for k in range(BPW // CH):
        lax.fori_loop(k * CH, (k + 1) * CH, row_body, 0)
        cnt_hs.append(pltpu.async_copy(
            cnt_v.at[pl.ds(k * CH, CH)],
            cnt_out.at[pl.ds(base + k * CH, CH)], sem2))
    ent_h.wait()
    for h in cnt_hs:
        h.wait()


def _tc_body(ent_ref, cnt_ref, tt_ref, wt_ref, pos_ref, type_ref,
             gam_ref, bet_ref, out_ref):
    cnt = cnt_ref[...]
    proj = lax.dot_general(ent_ref[...], wt_ref[...],
                           (((1,), (1,)), ((), ())),
                           preferred_element_type=F32)
    possum = jnp.dot(cnt, pos_ref[...], preferred_element_type=F32)
    n = jnp.sum(cnt, axis=-1, keepdims=True)
    pos = possum / jnp.maximum(n, 1e-7)
    t0 = type_ref[0:1, :]
    t1 = type_ref[1:2, :]
    tok = t0 + tt_ref[...].astype(F32) * (t1 - t0)
    emb = proj + pos + tok
    mu = jnp.mean(emb, axis=-1, keepdims=True)
    d = emb - mu
    var = jnp.mean(d * d, axis=-1, keepdims=True)
    out_ref[...] = d * lax.rsqrt(var + LN_EPS) * gam_ref[...] + bet_ref[...]


def _tc_call(ent, cnt, tti, wt, pos_table, type_table, gamma, beta):
    return pl.pallas_call(
        _tc_body,
        grid=(B // BLK,),
        in_specs=[
            pl.BlockSpec((BLK, EMB), lambda i: (i, 0)),
            pl.BlockSpec((BLK, NPOS), lambda i: (i, 0)),
            pl.BlockSpec((BLK, 1), lambda i: (i, 0)),
            pl.BlockSpec((HID, EMB), lambda i: (0, 0)),
            pl.BlockSpec((NPOS, HID), lambda i: (0, 0)),
            pl.BlockSpec((2, HID), lambda i: (0, 0)),
            pl.BlockSpec((1, HID), lambda i: (0, 0)),
            pl.BlockSpec((1, HID), lambda i: (0, 0)),
        ],
        out_specs=pl.BlockSpec((BLK, HID), lambda i: (i, 0)),
        out_shape=jax.ShapeDtypeStruct((B, HID), F32),
    )(ent, cnt, tti, wt, pos_table, type_table, gamma, beta)


def kernel(entity_ids, position_ids, token_type_ids, entity_table, dense_W,
           pos_table, type_table, gamma, beta):
    tti = token_type_ids.reshape(B, 1)
    ent_rows, cnt = _sc_gather_counts_fn()(entity_table, entity_ids,
                                           position_ids)
    out = _tc_call(ent_rows, cnt, tti, dense_W,
                   pos_table, type_table,
                   gamma.reshape(1, HID), beta.reshape(1, HID))
    return out.reshape(16, 128, HID)
